# baseline (speedup 1.0000x reference)
import functools

import numpy as np
import jax
import jax.numpy as jnp
from jax import lax
from jax.experimental import pallas as pl
from jax.experimental.pallas import tpu as pltpu
from jax.experimental.pallas import tpu_sc as plsc

N = 10000
D = 128
H1 = 8
E = 320000
NPAD = 10240
TW = 136
CH = 96
NW = 32
E_TOT = E + N
_nch = -(-E_TOT // (CH * NW))
NCHUNK = _nch + (_nch % 2)
NC0 = (2 * NCHUNK * 44 // 100) & ~1
NC1 = 2 * NCHUNK - NC0
E_PAD = NCHUNK * CH * NW
RPT = NPAD // 16


def _one16():
    return jnp.where(
        lax.broadcasted_iota(jnp.int32, (1, 16), 1) >= 8, 1.0, 0.0
    ).astype(jnp.float32)


def _edge_body(H, table, dtable, eds, edd, zacc, out,
               sx0, sx1, dx0, dx1, rows0, rows1, ad0, ad1, pb, acc,
               sr0, sr1, sa0, sa1, sis0, sis1, sid0, sid1):
    c = lax.axis_index("c")
    s = lax.axis_index("s")
    nc = jnp.where(c == 0, NC0, NC1)
    base = jnp.where(c == 0, s * NC0, 16 * NC0 + s * NC1)
    bufs = ((sx0, dx0, rows0, ad0, sr0, sa0, sis0, sid0),
            (sx1, dx1, rows1, ad1, sr1, sa1, sis1, sid1))

    def fetch_sidx(i, b):
        sx, sis = bufs[b][0], bufs[b][6]
        pltpu.async_copy(eds.at[pl.ds((base + i) * CH, CH)], sx, sis)

    def fetch_didx(i, b):
        dx, sid = bufs[b][1], bufs[b][7]
        pltpu.async_copy(edd.at[pl.ds((base + i) * CH, CH)], dx, sid)

    def wait_idx(b):
        sx, sis = bufs[b][0], bufs[b][6]
        dx, sid = bufs[b][1], bufs[b][7]
        pltpu.make_async_copy(eds.at[pl.ds(0, CH)], sx, sis).wait()
        pltpu.make_async_copy(edd.at[pl.ds(0, CH)], dx, sid).wait()

    def gathers_start(b):
        sx, dx, rows, ad, sr, sa = bufs[b][0:6]
        pltpu.async_copy(table.at[sx], rows, sr)
        pltpu.async_copy(dtable.at[dx], ad, sa)

    def compute(b, nxt):
        sx, dx, rows, ad, sr, sa, sis, sid = bufs[b]
        pltpu.make_async_copy(table.at[sx], rows, sr).wait()
        pltpu.make_async_copy(dtable.at[dx], ad, sa).wait()

        @pl.when(nxt < nc)
        def _():
            fetch_sidx(nxt, b)

        iota = lax.iota(jnp.int32, 16)
        idx8 = jnp.where(iota < 8, 8 + 7 % H, 8 + (iota - 8) % H)

        @plsc.parallel_loop(0, CH, 1, unroll=4)
        def edge(k):
            lo = rows[k, pl.ds(120, 16)]
            alpha = lo + ad[k, :]
            alpha = jnp.maximum(alpha, 0.0) + 0.2 * jnp.minimum(alpha, 0.0)
            w = jnp.exp(alpha)
            for cc in range(8):
                m = w.at[jnp.full((16,), 8 + cc % H, jnp.int32)].get(
                    mode="promise_in_bounds")
                pb[k, pl.ds(cc * 16, 16)] = rows[k, pl.ds(cc * 16, 16)] * m
            m8 = w.at[idx8].get(mode="promise_in_bounds")
            pb[k, pl.ds(120, 16)] = jnp.where(iota < 8, lo, 1.0) * m8

        pltpu.sync_copy(pb, acc.at[dx], add=True)

        @pl.when(nxt < nc)
        def _():
            fetch_didx(nxt, b)

    fetch_sidx(0, 0)
    fetch_didx(0, 0)
    fetch_sidx(1, 1)
    fetch_didx(1, 1)
    wait_idx(0)
    gathers_start(0)
    wait_idx(1)
    gathers_start(1)

    @pl.when(s == 0)
    def _():
        pltpu.sync_copy(zacc, acc)

    plsc.subcore_barrier()

    def outer(j, carry):
        i0 = 2 * j
        compute(0, i0 + 2)

        @pl.when(i0 + 2 < nc)
        def _():
            wait_idx(0)
            gathers_start(0)

        compute(1, i0 + 3)

        @pl.when(i0 + 3 < nc)
        def _():
            wait_idx(1)
            gathers_start(1)

        return carry

    lax.fori_loop(0, nc // 2, outer, 0)

    plsc.subcore_barrier()
    pltpu.sync_copy(acc.at[pl.ds(s * RPT, RPT)],
                    out.at[c, pl.ds(s * RPT, RPT)])


@functools.lru_cache(maxsize=None)
def _make_edge_call(H):
    mesh = plsc.VectorSubcoreMesh(core_axis_name="c", subcore_axis_name="s",
                                  num_cores=2, num_subcores=16)
    return pl.kernel(
        functools.partial(_edge_body, H),
        out_type=jax.ShapeDtypeStruct((2, NPAD, TW), jnp.float32),
        mesh=mesh,
        compiler_params=pltpu.CompilerParams(
            needs_layout_passes=False, use_tc_tiling_on_sc=False),
        scratch_types=[
            pltpu.VMEM((CH,), jnp.int32),
            pltpu.VMEM((CH,), jnp.int32),
            pltpu.VMEM((CH,), jnp.int32),
            pltpu.VMEM((CH,), jnp.int32),
            pltpu.VMEM((CH, TW), jnp.float32),
            pltpu.VMEM((CH, TW), jnp.float32),
            pltpu.VMEM((CH, 16), jnp.float32),
            pltpu.VMEM((CH, 16), jnp.float32),
            pltpu.VMEM((CH, TW), jnp.float32),
            pltpu.VMEM_SHARED((NPAD, TW), jnp.float32),
            pltpu.SemaphoreType.DMA,
            pltpu.SemaphoreType.DMA,
            pltpu.SemaphoreType.DMA,
            pltpu.SemaphoreType.DMA,
            pltpu.SemaphoreType.DMA,
            pltpu.SemaphoreType.DMA,
            pltpu.SemaphoreType.DMA,
            pltpu.SemaphoreType.DMA,
        ],
    )


_RB = 640
_RC = 400


def _nodeA_body(x_ref, w1_ref, a1e_ref, a1d_ref, t_ref, dt_ref):
    h = jnp.dot(x_ref[...], w1_ref[...], preferred_element_type=jnp.float32)
    t_ref[:, 0:128] = h
    t_ref[:, 128:136] = jnp.dot(h, a1e_ref[...],
                                preferred_element_type=jnp.float32)
    dt_ref[...] = jnp.dot(h, a1d_ref[...], preferred_element_type=jnp.float32)


def _nodeB_body(p_ref, x_ref, b1_ref, scale_ref, shift_ref, w2_ref,
                a2e_ref, a2d_ref, sel_ref, t_ref, dt_ref):
    sblk = p_ref[0, :, :] + p_ref[1, :, :]
    den = jnp.dot(sblk, sel_ref[...], preferred_element_type=jnp.float32)
    g = sblk[:, 0:128] / (den + 1e-16) + b1_ref[...]
    g = g * scale_ref[...] + shift_ref[...]
    g = jnp.maximum(g, 0.0) + x_ref[...]
    h2 = jnp.dot(g, w2_ref[...], preferred_element_type=jnp.float32)
    t_ref[:, 0:128] = h2
    t_ref[:, 128:136] = jnp.dot(h2, a2e_ref[...],
                                preferred_element_type=jnp.float32)
    dt_ref[...] = jnp.dot(h2, a2d_ref[...], preferred_element_type=jnp.float32)


def _nodeC_body(p_ref, b2_ref, lng_ref, lnb_ref, sel_ref, y_ref):
    sblk = p_ref[0, :, :] + p_ref[1, :, :]
    den = jnp.dot(sblk, sel_ref[...], preferred_element_type=jnp.float32)
    h2 = sblk[:, 0:128] / (den + 1e-16) + b2_ref[...]
    mu = jnp.mean(h2, axis=-1, keepdims=True)
    dv = h2 - mu
    var = jnp.mean(dv * dv, axis=-1, keepdims=True)
    y_ref[...] = dv * lax.rsqrt(var + 1e-5) * lng_ref[...] + lnb_ref[...]


def _full(shape):
    return pl.BlockSpec(shape, lambda i: tuple(0 for _ in shape))


_nodeA = pl.pallas_call(
    _nodeA_body,
    grid=(NPAD // _RB,),
    in_specs=[
        pl.BlockSpec((_RB, D), lambda i: (i, 0)),
        _full((D, D)), _full((D, 8)), _full((D, 16)),
    ],
    out_specs=[
        pl.BlockSpec((_RB, TW), lambda i: (i, 0)),
        pl.BlockSpec((_RB, 16), lambda i: (i, 0)),
    ],
    out_shape=[
        jax.ShapeDtypeStruct((NPAD, TW), jnp.float32),
        jax.ShapeDtypeStruct((NPAD, 16), jnp.float32),
    ],
)

_nodeB = pl.pallas_call(
    _nodeB_body,
    grid=(NPAD // _RB,),
    in_specs=[
        pl.BlockSpec((2, _RB, TW), lambda i: (0, i, 0)),
        pl.BlockSpec((_RB, D), lambda i: (i, 0)),
        _full((1, D)), _full((1, D)), _full((1, D)),
        _full((D, D)), _full((D, 8)), _full((D, 16)), _full((TW, D)),
    ],
    out_specs=[
        pl.BlockSpec((_RB, TW), lambda i: (i, 0)),
        pl.BlockSpec((_RB, 16), lambda i: (i, 0)),
    ],
    out_shape=[
        jax.ShapeDtypeStruct((NPAD, TW), jnp.float32),
        jax.ShapeDtypeStruct((NPAD, 16), jnp.float32),
    ],
)

_nodeC = pl.pallas_call(
    _nodeC_body,
    grid=(N // _RC,),
    in_specs=[
        pl.BlockSpec((2, _RC, TW), lambda i: (0, i, 0)),
        _full((1, D)), _full((1, D)), _full((1, D)), _full((TW, D)),
    ],
    out_specs=pl.BlockSpec((_RC, D), lambda i: (i, 0)),
    out_shape=jax.ShapeDtypeStruct((N, D), jnp.float32),
)


def kernel(x, edge_index, W1, a_src1, a_dst1, b1, bn_g, bn_b,
           W2, a_src2, a_dst2, b2, ln_g, ln_b):
    f32 = jnp.float32
    x_pad = jnp.concatenate([x, jnp.zeros((NPAD - N, D), f32)])
    loop = jnp.arange(N, dtype=jnp.int32)
    npad_e = E_PAD - E_TOT
    eds = jnp.concatenate(
        [edge_index[0], loop, jnp.zeros((npad_e,), jnp.int32)])
    edd = jnp.concatenate(
        [edge_index[1], loop, jnp.full((npad_e,), N, jnp.int32)])

    eye8 = jnp.eye(H1, dtype=f32)
    A1e = (a_src1[:, :, None] * eye8[:, None, :]).reshape(D, H1)
    A1d = jnp.concatenate(
        [jnp.zeros((D, 8), f32),
         (a_dst1[:, :, None] * eye8[:, None, :]).reshape(D, H1)], 1)
    A2e = jnp.zeros((D, 8), f32).at[:, 0].set(a_src2[0])
    A2d = jnp.zeros((D, 16), f32).at[:, 8].set(a_dst2[0])
    SEL1 = jnp.concatenate(
        [jnp.zeros((128, D), f32), jnp.repeat(eye8, 16, axis=1)], 0)
    SEL2 = jnp.concatenate(
        [jnp.zeros((128, D), f32), jnp.full((8, D), 0.125, f32)], 0)
    zacc = jnp.zeros((NPAD, TW), f32)
    bn_scale = (bn_g / jnp.sqrt(1.0 + 1e-5)).reshape(1, D)

    table1, dt1 = _nodeA(x_pad, W1, A1e, A1d)
    p1 = _make_edge_call(H1)(table1, dt1, eds, edd, zacc)
    table2, dt2 = _nodeB(p1, x_pad, b1.reshape(1, D), bn_scale,
                         bn_b.reshape(1, D), W2, A2e, A2d, SEL1)
    p2 = _make_edge_call(1)(table2, dt2, eds, edd, zacc)
    return _nodeC(p2, b2.reshape(1, D), ln_g.reshape(1, D),
                  ln_b.reshape(1, D), SEL2)


# speedup vs baseline: 1.0390x; 1.0390x over previous
import functools

import numpy as np
import jax
import jax.numpy as jnp
from jax import lax
from jax.experimental import pallas as pl
from jax.experimental.pallas import tpu as pltpu
from jax.experimental.pallas import tpu_sc as plsc

N = 10000
D = 128
H1 = 8
E = 320000
NPAD = 10240
TW = 136
CH = 96
NW = 32
E_TOT = E + N
_nch = -(-E_TOT // (CH * NW))
NCHUNK = _nch + (_nch % 2)
NC0 = NCHUNK
NC1 = 2 * NCHUNK - NC0
E_PAD = NCHUNK * CH * NW
RPT = NPAD // 16


def _one16():
    return jnp.where(
        lax.broadcasted_iota(jnp.int32, (1, 16), 1) >= 8, 1.0, 0.0
    ).astype(jnp.float32)


def _edge_body(H, table, dtable, eds, edd, zacc, out,
               sx0, sx1, dx0, dx1, rows0, rows1, ad0, ad1, pb, acc,
               sr0, sr1, sa0, sa1, sis0, sis1, sid0, sid1):
    c = lax.axis_index("c")
    s = lax.axis_index("s")
    nc = jnp.where(c == 0, NC0, NC1)
    base = jnp.where(c == 0, s * NC0, 16 * NC0 + s * NC1)
    bufs = ((sx0, dx0, rows0, ad0, sr0, sa0, sis0, sid0),
            (sx1, dx1, rows1, ad1, sr1, sa1, sis1, sid1))

    def fetch_sidx(i, b):
        sx, sis = bufs[b][0], bufs[b][6]
        pltpu.async_copy(eds.at[pl.ds((base + i) * CH, CH)], sx, sis)

    def fetch_didx(i, b):
        dx, sid = bufs[b][1], bufs[b][7]
        pltpu.async_copy(edd.at[pl.ds((base + i) * CH, CH)], dx, sid)

    def wait_idx(b):
        sx, sis = bufs[b][0], bufs[b][6]
        dx, sid = bufs[b][1], bufs[b][7]
        pltpu.make_async_copy(eds.at[pl.ds(0, CH)], sx, sis).wait()
        pltpu.make_async_copy(edd.at[pl.ds(0, CH)], dx, sid).wait()

    def gathers_start(b):
        sx, dx, rows, ad, sr, sa = bufs[b][0:6]
        pltpu.async_copy(table.at[sx], rows, sr)
        pltpu.async_copy(dtable.at[dx], ad, sa)

    def compute(b, nxt):
        sx, dx, rows, ad, sr, sa, sis, sid = bufs[b]
        pltpu.make_async_copy(table.at[sx], rows, sr).wait()
        pltpu.make_async_copy(dtable.at[dx], ad, sa).wait()

        @pl.when(nxt < nc)
        def _():
            fetch_sidx(nxt, b)

        iota = lax.iota(jnp.int32, 16)
        idx8 = jnp.where(iota < 8, 8 + 7 % H, 8 + (iota - 8) % H)

        @plsc.parallel_loop(0, CH, 1, unroll=4)
        def edge(k):
            lo = rows[k, pl.ds(120, 16)]
            alpha = lo + ad[k, :]
            alpha = jnp.maximum(alpha, 0.0) + 0.2 * jnp.minimum(alpha, 0.0)
            w = jnp.exp(alpha)
            for cc in range(8):
                m = w.at[jnp.full((16,), 8 + cc % H, jnp.int32)].get(
                    mode="promise_in_bounds")
                pb[k, pl.ds(cc * 16, 16)] = rows[k, pl.ds(cc * 16, 16)] * m
            m8 = w.at[idx8].get(mode="promise_in_bounds")
            pb[k, pl.ds(120, 16)] = jnp.where(iota < 8, lo, 1.0) * m8

        pltpu.sync_copy(pb, acc.at[dx], add=True)

        @pl.when(nxt < nc)
        def _():
            fetch_didx(nxt, b)

    fetch_sidx(0, 0)
    fetch_didx(0, 0)
    fetch_sidx(1, 1)
    fetch_didx(1, 1)
    wait_idx(0)
    gathers_start(0)
    wait_idx(1)
    gathers_start(1)

    @pl.when(s == 0)
    def _():
        pltpu.sync_copy(zacc, acc)

    plsc.subcore_barrier()

    def outer(j, carry):
        i0 = 2 * j
        compute(0, i0 + 2)

        @pl.when(i0 + 2 < nc)
        def _():
            wait_idx(0)
            gathers_start(0)

        compute(1, i0 + 3)

        @pl.when(i0 + 3 < nc)
        def _():
            wait_idx(1)
            gathers_start(1)

        return carry

    lax.fori_loop(0, nc // 2, outer, 0)

    plsc.subcore_barrier()
    pltpu.sync_copy(acc.at[pl.ds(s * RPT, RPT)],
                    out.at[c, pl.ds(s * RPT, RPT)])


@functools.lru_cache(maxsize=None)
def _make_edge_call(H):
    mesh = plsc.VectorSubcoreMesh(core_axis_name="c", subcore_axis_name="s",
                                  num_cores=2, num_subcores=16)
    return pl.kernel(
        functools.partial(_edge_body, H),
        out_type=jax.ShapeDtypeStruct((2, NPAD, TW), jnp.float32),
        mesh=mesh,
        compiler_params=pltpu.CompilerParams(
            needs_layout_passes=False, use_tc_tiling_on_sc=False),
        scratch_types=[
            pltpu.VMEM((CH,), jnp.int32),
            pltpu.VMEM((CH,), jnp.int32),
            pltpu.VMEM((CH,), jnp.int32),
            pltpu.VMEM((CH,), jnp.int32),
            pltpu.VMEM((CH, TW), jnp.float32),
            pltpu.VMEM((CH, TW), jnp.float32),
            pltpu.VMEM((CH, 16), jnp.float32),
            pltpu.VMEM((CH, 16), jnp.float32),
            pltpu.VMEM((CH, TW), jnp.float32),
            pltpu.VMEM_SHARED((NPAD, TW), jnp.float32),
            pltpu.SemaphoreType.DMA,
            pltpu.SemaphoreType.DMA,
            pltpu.SemaphoreType.DMA,
            pltpu.SemaphoreType.DMA,
            pltpu.SemaphoreType.DMA,
            pltpu.SemaphoreType.DMA,
            pltpu.SemaphoreType.DMA,
            pltpu.SemaphoreType.DMA,
        ],
    )


_RB = 640
_RC = 400


def _nodeA_body(x_ref, w1_ref, a1e_ref, a1d_ref, t_ref, dt_ref):
    h = jnp.dot(x_ref[...], w1_ref[...], preferred_element_type=jnp.float32)
    t_ref[:, 0:128] = h
    t_ref[:, 128:136] = jnp.dot(h, a1e_ref[...],
                                preferred_element_type=jnp.float32)
    dt_ref[...] = jnp.dot(h, a1d_ref[...], preferred_element_type=jnp.float32)


def _nodeB_body(p_ref, x_ref, b1_ref, scale_ref, shift_ref, w2_ref,
                a2e_ref, a2d_ref, sel_ref, t_ref, dt_ref):
    sblk = p_ref[0, :, :] + p_ref[1, :, :]
    den = jnp.dot(sblk, sel_ref[...], preferred_element_type=jnp.float32)
    g = sblk[:, 0:128] / (den + 1e-16) + b1_ref[...]
    g = g * scale_ref[...] + shift_ref[...]
    g = jnp.maximum(g, 0.0) + x_ref[...]
    h2 = jnp.dot(g, w2_ref[...], preferred_element_type=jnp.float32)
    t_ref[:, 0:128] = h2
    t_ref[:, 128:136] = jnp.dot(h2, a2e_ref[...],
                                preferred_element_type=jnp.float32)
    dt_ref[...] = jnp.dot(h2, a2d_ref[...], preferred_element_type=jnp.float32)


def _nodeC_body(p_ref, b2_ref, lng_ref, lnb_ref, sel_ref, y_ref):
    sblk = p_ref[0, :, :] + p_ref[1, :, :]
    den = jnp.dot(sblk, sel_ref[...], preferred_element_type=jnp.float32)
    h2 = sblk[:, 0:128] / (den + 1e-16) + b2_ref[...]
    mu = jnp.mean(h2, axis=-1, keepdims=True)
    dv = h2 - mu
    var = jnp.mean(dv * dv, axis=-1, keepdims=True)
    y_ref[...] = dv * lax.rsqrt(var + 1e-5) * lng_ref[...] + lnb_ref[...]


def _full(shape):
    return pl.BlockSpec(shape, lambda i: tuple(0 for _ in shape))


_nodeA = pl.pallas_call(
    _nodeA_body,
    grid=(NPAD // _RB,),
    in_specs=[
        pl.BlockSpec((_RB, D), lambda i: (i, 0)),
        _full((D, D)), _full((D, 8)), _full((D, 16)),
    ],
    out_specs=[
        pl.BlockSpec((_RB, TW), lambda i: (i, 0)),
        pl.BlockSpec((_RB, 16), lambda i: (i, 0)),
    ],
    out_shape=[
        jax.ShapeDtypeStruct((NPAD, TW), jnp.float32),
        jax.ShapeDtypeStruct((NPAD, 16), jnp.float32),
    ],
)

_nodeB = pl.pallas_call(
    _nodeB_body,
    grid=(NPAD // _RB,),
    in_specs=[
        pl.BlockSpec((2, _RB, TW), lambda i: (0, i, 0)),
        pl.BlockSpec((_RB, D), lambda i: (i, 0)),
        _full((1, D)), _full((1, D)), _full((1, D)),
        _full((D, D)), _full((D, 8)), _full((D, 16)), _full((TW, D)),
    ],
    out_specs=[
        pl.BlockSpec((_RB, TW), lambda i: (i, 0)),
        pl.BlockSpec((_RB, 16), lambda i: (i, 0)),
    ],
    out_shape=[
        jax.ShapeDtypeStruct((NPAD, TW), jnp.float32),
        jax.ShapeDtypeStruct((NPAD, 16), jnp.float32),
    ],
)

_nodeC = pl.pallas_call(
    _nodeC_body,
    grid=(N // _RC,),
    in_specs=[
        pl.BlockSpec((2, _RC, TW), lambda i: (0, i, 0)),
        _full((1, D)), _full((1, D)), _full((1, D)), _full((TW, D)),
    ],
    out_specs=pl.BlockSpec((_RC, D), lambda i: (i, 0)),
    out_shape=jax.ShapeDtypeStruct((N, D), jnp.float32),
)


def kernel(x, edge_index, W1, a_src1, a_dst1, b1, bn_g, bn_b,
           W2, a_src2, a_dst2, b2, ln_g, ln_b):
    f32 = jnp.float32
    x_pad = jnp.concatenate([x, jnp.zeros((NPAD - N, D), f32)])
    loop = jnp.arange(N, dtype=jnp.int32)
    npad_e = E_PAD - E_TOT
    eds = jnp.concatenate(
        [edge_index[0], loop, jnp.zeros((npad_e,), jnp.int32)])
    edd = jnp.concatenate(
        [edge_index[1], loop, jnp.full((npad_e,), N, jnp.int32)])

    eye8 = jnp.eye(H1, dtype=f32)
    A1e = (a_src1[:, :, None] * eye8[:, None, :]).reshape(D, H1)
    A1d = jnp.concatenate(
        [jnp.zeros((D, 8), f32),
         (a_dst1[:, :, None] * eye8[:, None, :]).reshape(D, H1)], 1)
    A2e = jnp.zeros((D, 8), f32).at[:, 0].set(a_src2[0])
    A2d = jnp.zeros((D, 16), f32).at[:, 8].set(a_dst2[0])
    SEL1 = jnp.concatenate(
        [jnp.zeros((128, D), f32), jnp.repeat(eye8, 16, axis=1)], 0)
    SEL2 = jnp.concatenate(
        [jnp.zeros((128, D), f32), jnp.full((8, D), 0.125, f32)], 0)
    zacc = jnp.zeros((NPAD, TW), f32)
    bn_scale = (bn_g / jnp.sqrt(1.0 + 1e-5)).reshape(1, D)

    table1, dt1 = _nodeA(x_pad, W1, A1e, A1d)
    p1 = _make_edge_call(H1)(table1, dt1, eds, edd, zacc)
    table2, dt2 = _nodeB(p1, x_pad, b1.reshape(1, D), bn_scale,
                         bn_b.reshape(1, D), W2, A2e, A2d, SEL1)
    p2 = _make_edge_call(1)(table2, dt2, eds, edd, zacc)
    return _nodeC(p2, b2.reshape(1, D), ln_g.reshape(1, D),
                  ln_b.reshape(1, D), SEL2)
